# baseline (device time: 575322 ns/iter reference)
import jax
import jax.numpy as jnp
from jax import lax
from jax.experimental import pallas as pl
from jax.experimental.pallas import tpu as pltpu

W = 32
M_PER = 128
N = 2048


def kernel(x, w_mat, scale_x, scale_w):
    m, k_per = x.shape
    _, n = w_mat.shape

    def body(x_ref, w_ref, sx_ref, sw_ref, out_ref,
             comm_ref, send_sems, recv_sems, credit_sem):
        my = lax.axis_index("i")
        left = lax.rem(my + W - 1, W)
        right = lax.rem(my + 1, W)

        barrier_sem = pltpu.get_barrier_semaphore()
        for nbr in (left, right):
            pl.semaphore_signal(
                barrier_sem, inc=1,
                device_id=(nbr,), device_id_type=pl.DeviceIdType.MESH,
            )
        pl.semaphore_wait(barrier_sem, 2)

        w_bf = w_ref[:, :].astype(jnp.bfloat16)

        def chunk_partial(c):
            xs = x_ref[pl.ds(c * M_PER, M_PER), :].astype(jnp.bfloat16)
            return jnp.dot(xs, w_bf, preferred_element_type=jnp.float32)

        comm_ref[1, :, :] = chunk_partial(lax.rem(my + W - 1, W))

        for s in range(1, W):
            send_slot = s % 2
            recv_slot = (s + 1) % 2
            if s >= 3:
                pl.semaphore_wait(credit_sem, 1)
            rdma = pltpu.make_async_remote_copy(
                src_ref=comm_ref.at[send_slot],
                dst_ref=comm_ref.at[recv_slot],
                send_sem=send_sems.at[send_slot],
                recv_sem=recv_sems.at[recv_slot],
                device_id=(right,),
                device_id_type=pl.DeviceIdType.MESH,
            )
            rdma.start()
            rdma.wait()
            j = lax.rem(my + 2 * W - s - 1, W)
            comm_ref[recv_slot, :, :] += chunk_partial(j)
            if 2 <= s <= W - 1 - 1:
                pl.semaphore_signal(
                    credit_sem, inc=1,
                    device_id=(left,), device_id_type=pl.DeviceIdType.MESH,
                )

        scale = sx_ref[0] * sw_ref[0]
        out_ref[:, :] = jnp.maximum(comm_ref[0, :, :] * scale, 0.0)

    return pl.pallas_call(
        body,
        out_shape=jax.ShapeDtypeStruct((M_PER, n), jnp.float32),
        in_specs=[
            pl.BlockSpec(memory_space=pltpu.VMEM),
            pl.BlockSpec(memory_space=pltpu.VMEM),
            pl.BlockSpec(memory_space=pltpu.SMEM),
            pl.BlockSpec(memory_space=pltpu.SMEM),
        ],
        out_specs=pl.BlockSpec(memory_space=pltpu.VMEM),
        scratch_shapes=[
            pltpu.VMEM((2, M_PER, n), jnp.float32),
            pltpu.SemaphoreType.DMA((2,)),
            pltpu.SemaphoreType.DMA((2,)),
            pltpu.SemaphoreType.REGULAR,
        ],
        compiler_params=pltpu.CompilerParams(collective_id=0),
    )(x, w_mat, scale_x, scale_w)


# device time: 405914 ns/iter; 1.4173x vs baseline; 1.4173x over previous
import jax
import jax.numpy as jnp
from jax import lax
from jax.experimental import pallas as pl
from jax.experimental.pallas import tpu as pltpu

W = 32
M_PER = 128
N = 2048
HN = N // 2
S = 4

COMM_DTYPE = jnp.float32


def kernel(x, w_mat, scale_x, scale_w):
    m, k_per = x.shape
    _, n = w_mat.shape

    def body(x_ref, w_ref, sx_ref, sw_ref, out_ref,
             comm_r, comm_l, send_sems_r, recv_sems_r, send_sems_l,
             recv_sems_l):
        my = lax.axis_index("i")
        left = lax.rem(my + W - 1, W)
        right = lax.rem(my + 1, W)

        barrier_sem = pltpu.get_barrier_semaphore()
        for nbr in (left, right):
            pl.semaphore_signal(
                barrier_sem, inc=1,
                device_id=(nbr,), device_id_type=pl.DeviceIdType.MESH,
            )
        pl.semaphore_wait(barrier_sem, 2)

        w_r = w_ref[:, 0:HN].astype(jnp.bfloat16)
        w_l = w_ref[:, HN:N].astype(jnp.bfloat16)

        def partial_r(c):
            xs = x_ref[pl.ds(c * M_PER, M_PER), :].astype(jnp.bfloat16)
            return jnp.dot(xs, w_r, preferred_element_type=jnp.float32)

        def partial_l(c):
            xs = x_ref[pl.ds(c * M_PER, M_PER), :].astype(jnp.bfloat16)
            return jnp.dot(xs, w_l, preferred_element_type=jnp.float32)

        comm_r[0, :, :] = partial_r(lax.rem(my + W - 1, W)).astype(COMM_DTYPE)
        comm_l[0, :, :] = partial_l(lax.rem(my + 1, W)).astype(COMM_DTYPE)

        for s in range(1, W):
            send_slot = (s - 1) % S
            recv_slot = s % S
            rdma_r = pltpu.make_async_remote_copy(
                src_ref=comm_r.at[send_slot],
                dst_ref=comm_r.at[recv_slot],
                send_sem=send_sems_r.at[recv_slot],
                recv_sem=recv_sems_r.at[recv_slot],
                device_id=(right,),
                device_id_type=pl.DeviceIdType.MESH,
            )
            rdma_l = pltpu.make_async_remote_copy(
                src_ref=comm_l.at[send_slot],
                dst_ref=comm_l.at[recv_slot],
                send_sem=send_sems_l.at[recv_slot],
                recv_sem=recv_sems_l.at[recv_slot],
                device_id=(left,),
                device_id_type=pl.DeviceIdType.MESH,
            )
            rdma_r.start()
            rdma_l.start()
            p_r = partial_r(lax.rem(my + 2 * W - s - 1, W))
            p_l = partial_l(lax.rem(my + s + 1, W))
            rdma_r.wait()
            rdma_l.wait()
            comm_r[recv_slot, :, :] = (
                comm_r[recv_slot, :, :].astype(jnp.float32) + p_r
            ).astype(COMM_DTYPE)
            comm_l[recv_slot, :, :] = (
                comm_l[recv_slot, :, :].astype(jnp.float32) + p_l
            ).astype(COMM_DTYPE)

        last = (W - 1) % S
        scale = sx_ref[0] * sw_ref[0]
        out_ref[:, 0:HN] = jnp.maximum(
            comm_r[last, :, :].astype(jnp.float32) * scale, 0.0)
        out_ref[:, HN:N] = jnp.maximum(
            comm_l[last, :, :].astype(jnp.float32) * scale, 0.0)

    return pl.pallas_call(
        body,
        out_shape=jax.ShapeDtypeStruct((M_PER, n), jnp.float32),
        in_specs=[
            pl.BlockSpec(memory_space=pltpu.VMEM),
            pl.BlockSpec(memory_space=pltpu.VMEM),
            pl.BlockSpec(memory_space=pltpu.SMEM),
            pl.BlockSpec(memory_space=pltpu.SMEM),
        ],
        out_specs=pl.BlockSpec(memory_space=pltpu.VMEM),
        scratch_shapes=[
            pltpu.VMEM((S, M_PER, HN), COMM_DTYPE),
            pltpu.VMEM((S, M_PER, HN), COMM_DTYPE),
            pltpu.SemaphoreType.DMA((S,)),
            pltpu.SemaphoreType.DMA((S,)),
            pltpu.SemaphoreType.DMA((S,)),
            pltpu.SemaphoreType.DMA((S,)),
        ],
        compiler_params=pltpu.CompilerParams(collective_id=0),
    )(x, w_mat, scale_x, scale_w)


# device time: 160604 ns/iter; 3.5822x vs baseline; 2.5274x over previous
import jax
import jax.numpy as jnp
from jax import lax
from jax.experimental import pallas as pl
from jax.experimental.pallas import tpu as pltpu

W = 32
M_PER = 128
N = 2048
HN = N // 2
S = 4

COMM_DTYPE = jnp.bfloat16

CYC_LIST = [0, 1, 2, 5, 6, 14, 13, 10, 9, 17, 18, 21, 22, 30, 29, 26,
            25, 24, 27, 28, 31, 23, 20, 19, 16, 8, 11, 12, 15, 7, 4, 3]
RANK_LIST = [0] * W
for _r, _p in enumerate(CYC_LIST):
    RANK_LIST[_p] = _r


def kernel(x, w_mat, scale_x, scale_w):
    m, k_per = x.shape
    _, n = w_mat.shape

    cyc = jnp.asarray(CYC_LIST, dtype=jnp.int32)
    rank = jnp.asarray(RANK_LIST, dtype=jnp.int32)
    my = lax.axis_index("i")
    rho = rank[my]
    nbrs = jnp.stack([cyc[(rho + 1) % W], cyc[(rho + W - 1) % W]])
    ts = jnp.arange(W, dtype=jnp.int32)
    idx_r = cyc[(rho - ts - 1) % W]
    idx_l = cyc[(rho + ts + 1) % W]

    def body(idx_r_ref, idx_l_ref, nbr_ref, x_ref, w_ref, sx_ref, sw_ref,
             out_ref, comm_r, comm_l, send_sems_r, recv_sems_r,
             send_sems_l, recv_sems_l):
        right = nbr_ref[0]
        left = nbr_ref[1]

        barrier_sem = pltpu.get_barrier_semaphore()
        for nbr in (left, right):
            pl.semaphore_signal(
                barrier_sem, inc=1,
                device_id=(nbr,), device_id_type=pl.DeviceIdType.MESH,
            )
        pl.semaphore_wait(barrier_sem, 2)

        w_r = w_ref[:, 0:HN].astype(jnp.bfloat16)
        w_l = w_ref[:, HN:N].astype(jnp.bfloat16)

        def partial_r(c):
            xs = x_ref[pl.ds(c * M_PER, M_PER), :].astype(jnp.bfloat16)
            return jnp.dot(xs, w_r, preferred_element_type=jnp.float32)

        def partial_l(c):
            xs = x_ref[pl.ds(c * M_PER, M_PER), :].astype(jnp.bfloat16)
            return jnp.dot(xs, w_l, preferred_element_type=jnp.float32)

        comm_r[0, :, :] = partial_r(idx_r_ref[0]).astype(COMM_DTYPE)
        comm_l[0, :, :] = partial_l(idx_l_ref[0]).astype(COMM_DTYPE)

        for s in range(1, W):
            send_slot = (s - 1) % S
            recv_slot = s % S
            rdma_r = pltpu.make_async_remote_copy(
                src_ref=comm_r.at[send_slot],
                dst_ref=comm_r.at[recv_slot],
                send_sem=send_sems_r.at[recv_slot],
                recv_sem=recv_sems_r.at[recv_slot],
                device_id=(right,),
                device_id_type=pl.DeviceIdType.MESH,
            )
            rdma_l = pltpu.make_async_remote_copy(
                src_ref=comm_l.at[send_slot],
                dst_ref=comm_l.at[recv_slot],
                send_sem=send_sems_l.at[recv_slot],
                recv_sem=recv_sems_l.at[recv_slot],
                device_id=(left,),
                device_id_type=pl.DeviceIdType.MESH,
            )
            rdma_r.start()
            rdma_l.start()
            p_r = partial_r(idx_r_ref[s])
            p_l = partial_l(idx_l_ref[s])
            rdma_r.wait()
            rdma_l.wait()
            comm_r[recv_slot, :, :] = (
                comm_r[recv_slot, :, :].astype(jnp.float32) + p_r
            ).astype(COMM_DTYPE)
            comm_l[recv_slot, :, :] = (
                comm_l[recv_slot, :, :].astype(jnp.float32) + p_l
            ).astype(COMM_DTYPE)

        last = (W - 1) % S
        scale = sx_ref[0] * sw_ref[0]
        out_ref[:, 0:HN] = jnp.maximum(
            comm_r[last, :, :].astype(jnp.float32) * scale, 0.0)
        out_ref[:, HN:N] = jnp.maximum(
            comm_l[last, :, :].astype(jnp.float32) * scale, 0.0)

    return pl.pallas_call(
        body,
        out_shape=jax.ShapeDtypeStruct((M_PER, n), jnp.float32),
        in_specs=[
            pl.BlockSpec(memory_space=pltpu.SMEM),
            pl.BlockSpec(memory_space=pltpu.SMEM),
            pl.BlockSpec(memory_space=pltpu.SMEM),
            pl.BlockSpec(memory_space=pltpu.VMEM),
            pl.BlockSpec(memory_space=pltpu.VMEM),
            pl.BlockSpec(memory_space=pltpu.SMEM),
            pl.BlockSpec(memory_space=pltpu.SMEM),
        ],
        out_specs=pl.BlockSpec(memory_space=pltpu.VMEM),
        scratch_shapes=[
            pltpu.VMEM((S, M_PER, HN), COMM_DTYPE),
            pltpu.VMEM((S, M_PER, HN), COMM_DTYPE),
            pltpu.SemaphoreType.DMA((S,)),
            pltpu.SemaphoreType.DMA((S,)),
            pltpu.SemaphoreType.DMA((S,)),
            pltpu.SemaphoreType.DMA((S,)),
        ],
        compiler_params=pltpu.CompilerParams(collective_id=0),
    )(idx_r, idx_l, nbrs, x, w_mat, scale_x, scale_w)


# device time: 114873 ns/iter; 5.0083x vs baseline; 1.3981x over previous
import jax
import jax.numpy as jnp
from jax import lax
from jax.experimental import pallas as pl
from jax.experimental.pallas import tpu as pltpu

W = 32
M_PER = 128
N = 2048
HN = N // 2
S = 8
Q = 2
QW = HN // Q

COMM_DTYPE = jnp.bfloat16

CYC_LIST = [0, 1, 2, 5, 6, 14, 13, 10, 9, 17, 18, 21, 22, 30, 29, 26,
            25, 24, 27, 28, 31, 23, 20, 19, 16, 8, 11, 12, 15, 7, 4, 3]
RANK_LIST = [0] * W
for _r, _p in enumerate(CYC_LIST):
    RANK_LIST[_p] = _r


def kernel(x, w_mat, scale_x, scale_w):
    m, k_per = x.shape
    _, n = w_mat.shape

    cyc = jnp.asarray(CYC_LIST, dtype=jnp.int32)
    rank = jnp.asarray(RANK_LIST, dtype=jnp.int32)
    my = lax.axis_index("i")
    rho = rank[my]
    nbrs = jnp.stack([cyc[(rho + 1) % W], cyc[(rho + W - 1) % W]])
    ts = jnp.arange(W, dtype=jnp.int32)
    idx_r = cyc[(rho - ts - 1) % W]
    idx_l = cyc[(rho + ts + 1) % W]

    def body(idx_r_ref, idx_l_ref, nbr_ref, x_ref, w_ref, sx_ref, sw_ref,
             out_ref, comm_r, comm_l, send_sems_r, recv_sems_r,
             send_sems_l, recv_sems_l):
        right = nbr_ref[0]
        left = nbr_ref[1]

        barrier_sem = pltpu.get_barrier_semaphore()
        for nbr in (left, right):
            pl.semaphore_signal(
                barrier_sem, inc=1,
                device_id=(nbr,), device_id_type=pl.DeviceIdType.MESH,
            )
        pl.semaphore_wait(barrier_sem, 2)

        w_r = w_ref[:, 0:HN].astype(jnp.bfloat16)
        w_l = w_ref[:, HN:N].astype(jnp.bfloat16)

        def partial_r(c):
            xs = x_ref[pl.ds(c * M_PER, M_PER), :].astype(jnp.bfloat16)
            return jnp.dot(xs, w_r, preferred_element_type=jnp.float32)

        def partial_l(c):
            xs = x_ref[pl.ds(c * M_PER, M_PER), :].astype(jnp.bfloat16)
            return jnp.dot(xs, w_l, preferred_element_type=jnp.float32)

        def mk(buf, ssems, rsems, h, q, dev):
            cols = pl.ds(q * QW, QW)
            return pltpu.make_async_remote_copy(
                src_ref=buf.at[(h - 1) % S, :, cols],
                dst_ref=buf.at[h % S, :, cols],
                send_sem=ssems.at[h % S, q],
                recv_sem=rsems.at[h % S, q],
                device_id=(dev,),
                device_id_type=pl.DeviceIdType.MESH,
            )

        comm_r[0, :, :] = partial_r(idx_r_ref[0]).astype(COMM_DTYPE)
        comm_l[0, :, :] = partial_l(idx_l_ref[0]).astype(COMM_DTYPE)
        for q in range(Q):
            mk(comm_r, send_sems_r, recv_sems_r, 1, q, right).start()
            mk(comm_l, send_sems_l, recv_sems_l, 1, q, left).start()
        p_r = partial_r(idx_r_ref[1])
        p_l = partial_l(idx_l_ref[1])

        for s in range(1, W):
            slot = s % S
            for q in range(Q):
                cs = slice(q * QW, (q + 1) * QW)
                rr = mk(comm_r, send_sems_r, recv_sems_r, s, q, right)
                rr.wait_recv()
                comm_r[slot, :, cs] = (
                    comm_r[slot, :, cs].astype(jnp.float32) + p_r[:, cs]
                ).astype(COMM_DTYPE)
                if s < W - 1:
                    if s >= S:
                        mk(comm_r, send_sems_r, recv_sems_r,
                           s + 1 - S, q, right).wait_send()
                    mk(comm_r, send_sems_r, recv_sems_r,
                       s + 1, q, right).start()
                ll = mk(comm_l, send_sems_l, recv_sems_l, s, q, left)
                ll.wait_recv()
                comm_l[slot, :, cs] = (
                    comm_l[slot, :, cs].astype(jnp.float32) + p_l[:, cs]
                ).astype(COMM_DTYPE)
                if s < W - 1:
                    if s >= S:
                        mk(comm_l, send_sems_l, recv_sems_l,
                           s + 1 - S, q, left).wait_send()
                    mk(comm_l, send_sems_l, recv_sems_l,
                       s + 1, q, left).start()
            if s < W - 1:
                p_r = partial_r(idx_r_ref[s + 1])
                p_l = partial_l(idx_l_ref[s + 1])

        for h in range(W - S, W):
            for q in range(Q):
                mk(comm_r, send_sems_r, recv_sems_r, h, q, right).wait_send()
                mk(comm_l, send_sems_l, recv_sems_l, h, q, left).wait_send()

        last = (W - 1) % S
        scale = sx_ref[0] * sw_ref[0]
        out_ref[:, 0:HN] = jnp.maximum(
            comm_r[last, :, :].astype(jnp.float32) * scale, 0.0)
        out_ref[:, HN:N] = jnp.maximum(
            comm_l[last, :, :].astype(jnp.float32) * scale, 0.0)

    return pl.pallas_call(
        body,
        out_shape=jax.ShapeDtypeStruct((M_PER, n), jnp.float32),
        in_specs=[
            pl.BlockSpec(memory_space=pltpu.SMEM),
            pl.BlockSpec(memory_space=pltpu.SMEM),
            pl.BlockSpec(memory_space=pltpu.SMEM),
            pl.BlockSpec(memory_space=pltpu.VMEM),
            pl.BlockSpec(memory_space=pltpu.VMEM),
            pl.BlockSpec(memory_space=pltpu.SMEM),
            pl.BlockSpec(memory_space=pltpu.SMEM),
        ],
        out_specs=pl.BlockSpec(memory_space=pltpu.VMEM),
        scratch_shapes=[
            pltpu.VMEM((S, M_PER, HN), COMM_DTYPE),
            pltpu.VMEM((S, M_PER, HN), COMM_DTYPE),
            pltpu.SemaphoreType.DMA((S, Q)),
            pltpu.SemaphoreType.DMA((S, Q)),
            pltpu.SemaphoreType.DMA((S, Q)),
            pltpu.SemaphoreType.DMA((S, Q)),
        ],
        compiler_params=pltpu.CompilerParams(collective_id=0),
    )(idx_r, idx_l, nbrs, x, w_mat, scale_x, scale_w)


# device time: 104561 ns/iter; 5.5023x vs baseline; 1.0986x over previous
import jax
import jax.numpy as jnp
from jax import lax
from jax.experimental import pallas as pl
from jax.experimental.pallas import tpu as pltpu

W = 32
M_PER = 128
N = 2048
HN = N // 2
S = 8
Q = 4
QW = HN // Q

COMM_DTYPE = jnp.bfloat16

CYC_LIST = [0, 1, 2, 5, 6, 14, 13, 10, 9, 17, 18, 21, 22, 30, 29, 26,
            25, 24, 27, 28, 31, 23, 20, 19, 16, 8, 11, 12, 15, 7, 4, 3]
RANK_LIST = [0] * W
for _r, _p in enumerate(CYC_LIST):
    RANK_LIST[_p] = _r


def kernel(x, w_mat, scale_x, scale_w):
    m, k_per = x.shape
    _, n = w_mat.shape

    cyc = jnp.asarray(CYC_LIST, dtype=jnp.int32)
    rank = jnp.asarray(RANK_LIST, dtype=jnp.int32)
    my = lax.axis_index("i")
    rho = rank[my]
    nbrs = jnp.stack([cyc[(rho + 1) % W], cyc[(rho + W - 1) % W]])
    ts = jnp.arange(W, dtype=jnp.int32)
    idx_r = cyc[(rho - ts - 1) % W]
    idx_l = cyc[(rho + ts + 1) % W]

    def body(idx_r_ref, idx_l_ref, nbr_ref, x_ref, w_ref, sx_ref, sw_ref,
             out_ref, comm_r, comm_l, send_sems_r, recv_sems_r,
             send_sems_l, recv_sems_l):
        right = nbr_ref[0]
        left = nbr_ref[1]

        barrier_sem = pltpu.get_barrier_semaphore()
        for nbr in (left, right):
            pl.semaphore_signal(
                barrier_sem, inc=1,
                device_id=(nbr,), device_id_type=pl.DeviceIdType.MESH,
            )
        pl.semaphore_wait(barrier_sem, 2)

        w_r = w_ref[:, 0:HN].astype(jnp.bfloat16)
        w_l = w_ref[:, HN:N].astype(jnp.bfloat16)

        def partial_r(c):
            xs = x_ref[pl.ds(c * M_PER, M_PER), :].astype(jnp.bfloat16)
            return jnp.dot(xs, w_r, preferred_element_type=jnp.float32)

        def partial_l(c):
            xs = x_ref[pl.ds(c * M_PER, M_PER), :].astype(jnp.bfloat16)
            return jnp.dot(xs, w_l, preferred_element_type=jnp.float32)

        def mk(buf, ssems, rsems, h, q, dev):
            cols = pl.ds(q * QW, QW)
            return pltpu.make_async_remote_copy(
                src_ref=buf.at[(h - 1) % S, :, cols],
                dst_ref=buf.at[h % S, :, cols],
                send_sem=ssems.at[h % S, q],
                recv_sem=rsems.at[h % S, q],
                device_id=(dev,),
                device_id_type=pl.DeviceIdType.MESH,
            )

        comm_r[0, :, :] = partial_r(idx_r_ref[0]).astype(COMM_DTYPE)
        comm_l[0, :, :] = partial_l(idx_l_ref[0]).astype(COMM_DTYPE)
        for q in range(Q):
            mk(comm_r, send_sems_r, recv_sems_r, 1, q, right).start()
            mk(comm_l, send_sems_l, recv_sems_l, 1, q, left).start()
        p_r = partial_r(idx_r_ref[1])
        p_l = partial_l(idx_l_ref[1])

        for s in range(1, W):
            slot = s % S
            for q in range(Q):
                cs = slice(q * QW, (q + 1) * QW)
                rr = mk(comm_r, send_sems_r, recv_sems_r, s, q, right)
                rr.wait_recv()
                comm_r[slot, :, cs] = (
                    comm_r[slot, :, cs].astype(jnp.float32) + p_r[:, cs]
                ).astype(COMM_DTYPE)
                if s < W - 1:
                    if s >= S:
                        mk(comm_r, send_sems_r, recv_sems_r,
                           s + 1 - S, q, right).wait_send()
                    mk(comm_r, send_sems_r, recv_sems_r,
                       s + 1, q, right).start()
                ll = mk(comm_l, send_sems_l, recv_sems_l, s, q, left)
                ll.wait_recv()
                comm_l[slot, :, cs] = (
                    comm_l[slot, :, cs].astype(jnp.float32) + p_l[:, cs]
                ).astype(COMM_DTYPE)
                if s < W - 1:
                    if s >= S:
                        mk(comm_l, send_sems_l, recv_sems_l,
                           s + 1 - S, q, left).wait_send()
                    mk(comm_l, send_sems_l, recv_sems_l,
                       s + 1, q, left).start()
            if s < W - 1:
                p_r = partial_r(idx_r_ref[s + 1])
                p_l = partial_l(idx_l_ref[s + 1])

        for h in range(W - S, W):
            for q in range(Q):
                mk(comm_r, send_sems_r, recv_sems_r, h, q, right).wait_send()
                mk(comm_l, send_sems_l, recv_sems_l, h, q, left).wait_send()

        last = (W - 1) % S
        scale = sx_ref[0] * sw_ref[0]
        out_ref[:, 0:HN] = jnp.maximum(
            comm_r[last, :, :].astype(jnp.float32) * scale, 0.0)
        out_ref[:, HN:N] = jnp.maximum(
            comm_l[last, :, :].astype(jnp.float32) * scale, 0.0)

    return pl.pallas_call(
        body,
        out_shape=jax.ShapeDtypeStruct((M_PER, n), jnp.float32),
        in_specs=[
            pl.BlockSpec(memory_space=pltpu.SMEM),
            pl.BlockSpec(memory_space=pltpu.SMEM),
            pl.BlockSpec(memory_space=pltpu.SMEM),
            pl.BlockSpec(memory_space=pltpu.VMEM),
            pl.BlockSpec(memory_space=pltpu.VMEM),
            pl.BlockSpec(memory_space=pltpu.SMEM),
            pl.BlockSpec(memory_space=pltpu.SMEM),
        ],
        out_specs=pl.BlockSpec(memory_space=pltpu.VMEM),
        scratch_shapes=[
            pltpu.VMEM((S, M_PER, HN), COMM_DTYPE),
            pltpu.VMEM((S, M_PER, HN), COMM_DTYPE),
            pltpu.SemaphoreType.DMA((S, Q)),
            pltpu.SemaphoreType.DMA((S, Q)),
            pltpu.SemaphoreType.DMA((S, Q)),
            pltpu.SemaphoreType.DMA((S, Q)),
        ],
        compiler_params=pltpu.CompilerParams(collective_id=0),
    )(idx_r, idx_l, nbrs, x, w_mat, scale_x, scale_w)
